# baseline (speedup 1.0000x reference)
import functools
import jax
import jax.numpy as jnp
from jax import lax
from jax.experimental import pallas as pl
from jax.experimental.pallas import tpu as pltpu
from jax.experimental.pallas import tpu_sc as plsc

_B, _S, _D, _U = 64, 8192, 128, 16
_NB = 2
_CHT = 248
_NT = 16
_ROWS_T = [_CHT] * _NT
_SPLIT = _CHT * _NT
_CHS = 248
_REM = _S - _U - _SPLIT
_NSF = _REM // _CHS
_ROWS_S = [_CHS] * _NSF + ([_REM - _NSF * _CHS] if _REM % _CHS else [])


def _make_jobs(rows, base_off, with_update):
    jobs = []
    for b_i in range(2):
        off = base_off
        for r in rows:
            jobs.append((b_i, 0, off, r))
            off += r
        if with_update:
            jobs.append((b_i, 1, 0, _U))
    return jobs


def _sc_body(cache_hbm, update_hbm, out_hbm, tbuf, sbuf,
             t_in_sems, t_out_sems, s_in_sems, s_out_sems):
    c = lax.axis_index("c")
    s = lax.axis_index("s")
    wid = c * 16 + s

    jobs_t = _make_jobs(_ROWS_T, 0, True)
    jobs_s = _make_jobs(_ROWS_S, _SPLIT, False)

    def mk_ops(jobs, bufref, tile_sliced, in_sems, out_sems):
        def src_of(j):
            b_i, kind, off, r = jobs[j]
            b = wid * 2 + b_i
            if kind == 0:
                return cache_hbm.at[b, pl.ds(_U + off, r), :]
            return update_hbm.at[b, :, :]

        def dst_of(j):
            b_i, kind, off, r = jobs[j]
            b = wid * 2 + b_i
            if kind == 0:
                return out_hbm.at[b, pl.ds(off, r), :]
            return out_hbm.at[b, pl.ds(_S - _U, _U), :]

        def bufslice(j):
            r = jobs[j][3]
            if tile_sliced:
                return bufref.at[s, j % _NB, pl.ds(0, r), :]
            return bufref.at[j % _NB, pl.ds(0, r), :]

        def start_in(j):
            pltpu.make_async_copy(src_of(j), bufslice(j), in_sems.at[j % _NB]).start()

        def wait_in(j):
            pltpu.make_async_copy(src_of(j), bufslice(j), in_sems.at[j % _NB]).wait()

        def start_out(j):
            pltpu.make_async_copy(bufslice(j), dst_of(j), out_sems.at[j % _NB]).start()

        def wait_out(j):
            pltpu.make_async_copy(bufslice(j), dst_of(j), out_sems.at[j % _NB]).wait()

        return start_in, wait_in, start_out, wait_out

    t_ops = mk_ops(jobs_t, tbuf, False, t_in_sems, t_out_sems)
    s_ops = mk_ops(jobs_s, sbuf, True, s_in_sems, s_out_sems)
    JT, JS = len(jobs_t), len(jobs_s)

    def step(ops, j, J):
        start_in, wait_in, start_out, wait_out = ops
        if j + _NB - 1 < J:
            if j >= 1:
                wait_out(j - 1)
            start_in(j + _NB - 1)
        wait_in(j)
        start_out(j)

    for j in range(_NB - 1):
        t_ops[0](j)
        s_ops[0](j)
    for j in range(max(JT, JS)):
        if j < JT:
            step(t_ops, j, JT)
        if j < JS:
            step(s_ops, j, JS)
    for j in range(max(0, JT - _NB), JT):
        t_ops[3](j)
    for j in range(max(0, JS - _NB), JS):
        s_ops[3](j)


def kernel(cache, update):
    mesh = plsc.VectorSubcoreMesh(core_axis_name="c", subcore_axis_name="s")
    k = functools.partial(
        pl.kernel,
        mesh=mesh,
        out_type=jax.ShapeDtypeStruct((_B, _S, _D), jnp.float32),
        scratch_types=[
            pltpu.VMEM((_NB, _CHT, _D), jnp.float32),
            pltpu.VMEM_SHARED((16, _NB, _CHS, _D), jnp.float32),
            pltpu.SemaphoreType.DMA((_NB,)),
            pltpu.SemaphoreType.DMA((_NB,)),
            pltpu.SemaphoreType.DMA((_NB,)),
            pltpu.SemaphoreType.DMA((_NB,)),
        ],
    )(_sc_body)
    return k(cache, update)


# speedup vs baseline: 1.0231x; 1.0231x over previous
import functools
import jax
import jax.numpy as jnp
from jax import lax
from jax.experimental import pallas as pl
from jax.experimental.pallas import tpu as pltpu
from jax.experimental.pallas import tpu_sc as plsc

_B, _S, _D, _U = 64, 8192, 128, 16
_NB = 2
_CH = 504
_NFULL = (_S - _U) // _CH
_TAIL = (_S - _U) - _NFULL * _CH
_ROWS = [_CH] * _NFULL + ([_TAIL] if _TAIL else [])


def _sc_body(cache_hbm, update_hbm, out_hbm, buf, in_sems, out_sems):
    c = lax.axis_index("c")
    s = lax.axis_index("s")
    wid = c * 16 + s

    jobs = []
    for b_i in range(2):
        for k, r in enumerate(_ROWS):
            jobs.append((b_i, 0, k * _CH, r))
        jobs.append((b_i, 1, 0, _U))
    J = len(jobs)

    def src_of(j):
        b_i, kind, off, r = jobs[j]
        b = wid * 2 + b_i
        if kind == 0:
            return cache_hbm.at[b, pl.ds(_U + off, r), :]
        return update_hbm.at[b, :, :]

    def dst_of(j):
        b_i, kind, off, r = jobs[j]
        b = wid * 2 + b_i
        if kind == 0:
            return out_hbm.at[b, pl.ds(off, r), :]
        return out_hbm.at[b, pl.ds(_S - _U, _U), :]

    def bufslice(j):
        r = jobs[j][3]
        return buf.at[s, j % _NB, pl.ds(0, r), :]

    def start_in(j):
        pltpu.make_async_copy(src_of(j), bufslice(j), in_sems.at[j % _NB]).start()

    def wait_in(j):
        pltpu.make_async_copy(src_of(j), bufslice(j), in_sems.at[j % _NB]).wait()

    def start_out(j):
        pltpu.make_async_copy(bufslice(j), dst_of(j), out_sems.at[j % _NB]).start()

    def wait_out(j):
        pltpu.make_async_copy(bufslice(j), dst_of(j), out_sems.at[j % _NB]).wait()

    for j in range(_NB - 1):
        start_in(j)
    for j in range(J):
        if j + _NB - 1 < J:
            if j >= 1:
                wait_out(j - 1)
            start_in(j + _NB - 1)
        wait_in(j)
        start_out(j)
    for j in range(max(0, J - _NB), J):
        wait_out(j)


def kernel(cache, update):
    mesh = plsc.VectorSubcoreMesh(core_axis_name="c", subcore_axis_name="s")
    k = functools.partial(
        pl.kernel,
        mesh=mesh,
        out_type=jax.ShapeDtypeStruct((_B, _S, _D), jnp.float32),
        scratch_types=[
            pltpu.VMEM_SHARED((16, _NB, _CH, _D), jnp.float32),
            pltpu.SemaphoreType.DMA((_NB,)),
            pltpu.SemaphoreType.DMA((_NB,)),
        ],
    )(_sc_body)
    return k(cache, update)


# speedup vs baseline: 1.0259x; 1.0027x over previous
import functools
import jax
import jax.numpy as jnp
from jax import lax
from jax.experimental import pallas as pl
from jax.experimental.pallas import tpu as pltpu
from jax.experimental.pallas import tpu_sc as plsc

_B, _S, _D, _U = 64, 8192, 128, 16
_NB = 3
_CH = 336
_NFULL = (_S - _U) // _CH
_TAIL = (_S - _U) - _NFULL * _CH
_ROWS = [_CH] * _NFULL + ([_TAIL] if _TAIL else [])


def _sc_body(cache_hbm, update_hbm, out_hbm, buf, in_sems, out_sems):
    c = lax.axis_index("c")
    s = lax.axis_index("s")
    wid = c * 16 + s

    jobs = []
    for b_i in range(2):
        for k, r in enumerate(_ROWS):
            jobs.append((b_i, 0, k * _CH, r))
        jobs.append((b_i, 1, 0, _U))
    J = len(jobs)

    def src_of(j):
        b_i, kind, off, r = jobs[j]
        b = wid * 2 + b_i
        if kind == 0:
            return cache_hbm.at[b, pl.ds(_U + off, r), :]
        return update_hbm.at[b, :, :]

    def dst_of(j):
        b_i, kind, off, r = jobs[j]
        b = wid * 2 + b_i
        if kind == 0:
            return out_hbm.at[b, pl.ds(off, r), :]
        return out_hbm.at[b, pl.ds(_S - _U, _U), :]

    def bufslice(j):
        r = jobs[j][3]
        return buf.at[s, j % _NB, pl.ds(0, r), :]

    def start_in(j):
        pltpu.make_async_copy(src_of(j), bufslice(j), in_sems.at[j % _NB]).start()

    def wait_in(j):
        pltpu.make_async_copy(src_of(j), bufslice(j), in_sems.at[j % _NB]).wait()

    def start_out(j):
        pltpu.make_async_copy(bufslice(j), dst_of(j), out_sems.at[j % _NB]).start()

    def wait_out(j):
        pltpu.make_async_copy(bufslice(j), dst_of(j), out_sems.at[j % _NB]).wait()

    for j in range(_NB - 1):
        start_in(j)
    for j in range(J):
        if j + _NB - 1 < J:
            if j >= 1:
                wait_out(j - 1)
            start_in(j + _NB - 1)
        wait_in(j)
        start_out(j)
    for j in range(max(0, J - _NB), J):
        wait_out(j)


def kernel(cache, update):
    mesh = plsc.VectorSubcoreMesh(core_axis_name="c", subcore_axis_name="s")
    k = functools.partial(
        pl.kernel,
        mesh=mesh,
        out_type=jax.ShapeDtypeStruct((_B, _S, _D), jnp.float32),
        scratch_types=[
            pltpu.VMEM_SHARED((16, _NB, _CH, _D), jnp.float32),
            pltpu.SemaphoreType.DMA((_NB,)),
            pltpu.SemaphoreType.DMA((_NB,)),
        ],
    )(_sc_body)
    return k(cache, update)
